# baseline (speedup 1.0000x reference)
import functools

import jax
import jax.numpy as jnp
from jax import lax
from jax.experimental import pallas as pl
from jax.experimental.pallas import tpu as pltpu
from jax.experimental.pallas import tpu_sc as plsc

_L0 = 1000
_GROUP = 18432
_BLOCK = 2304
_MAXN = 512
_NGROUP = 8
_LANES = 16
_NROWS = 2 * _NGROUP


def _body(in_hbm, *refs):
    out0 = refs[0]
    outs = refs[1:1 + _NGROUP]
    rows_v = refs[1 + _NGROUP]
    l0_v = refs[2 + _NGROUP]
    sem_in = refs[3 + _NGROUP]
    sem_out = refs[4 + _NGROUP]
    sem_l0 = refs[5 + _NGROUP]

    wid = lax.axis_index("s") * 2 + lax.axis_index("c")

    neg_inf = jnp.full((_LANES,), -jnp.inf, dtype=jnp.float32)

    def _fire(i, _):
        t = lax.rem(i, 2)
        g = lax.div(i, 2)
        lr = wid + t * (63 - 2 * wid)
        m = lax.rem(lr, 8)
        blk = lax.div(lr, 8)
        in_off = _L0 + g * _GROUP + blk * _BLOCK + 32 * m * (m + 1)
        for k in range(8):
            @pl.when(m == k)
            def _(k=k):
                nn = 64 * (k + 1)
                pltpu.async_copy(in_hbm.at[pl.ds(in_off, nn)],
                                 rows_v.at[pl.ds(i * _MAXN, nn)], sem_in)
        return 0

    lax.fori_loop(0, _NROWS, _fire, 0)

    @pl.when(wid == 0)
    def _():
        pltpu.async_copy(in_hbm.at[pl.ds(0, _L0)], l0_v, sem_l0)

    pltpu.make_async_copy(in_hbm.at[pl.ds(0, _NGROUP * 576)],
                          rows_v.at[pl.ds(0, _NGROUP * 576)], sem_in).wait()

    @pl.when(wid == 0)
    def _():
        pltpu.make_async_copy(in_hbm.at[pl.ds(0, _L0)], l0_v, sem_l0).wait()
        pltpu.async_copy(l0_v, out0.at[0], sem_l0)

    for g in range(_NGROUP):
        for t in range(2):
            lr = wid + t * (63 - 2 * wid)
            m = lax.rem(lr, 8)
            base = (2 * g + t) * _MAXN

            def _pad64(c, _, base=base):
                for k in range(4):
                    rows_v[pl.ds(base + c * 64 + k * _LANES,
                                 _LANES)] = neg_inf
                return 0

            lax.fori_loop(m + 1, 8, _pad64, 0)
            pltpu.async_copy(rows_v.at[pl.ds(base, _MAXN)],
                             outs[g].at[lr], sem_out)

    pltpu.make_async_copy(in_hbm.at[pl.ds(0, _NROWS * _MAXN)],
                          rows_v, sem_out).wait()

    @pl.when(wid == 0)
    def _():
        pltpu.make_async_copy(in_hbm.at[pl.ds(0, _L0)], l0_v, sem_l0).wait()


_OUT_TYPE = (
    (jax.ShapeDtypeStruct((1, _L0), jnp.float32),)
    + tuple(jax.ShapeDtypeStruct((64, _MAXN), jnp.float32)
            for _ in range(_NGROUP))
)

_sc_interpret = functools.partial(
    pl.kernel,
    mesh=plsc.VectorSubcoreMesh(core_axis_name="c", subcore_axis_name="s"),
    out_type=_OUT_TYPE,
    scratch_types=[
        pltpu.VMEM((_NROWS * _MAXN,), jnp.float32),
        pltpu.VMEM((_L0,), jnp.float32),
        pltpu.SemaphoreType.DMA,
        pltpu.SemaphoreType.DMA,
        pltpu.SemaphoreType.DMA,
    ],
)(_body)


def kernel(logits):
    return _sc_interpret(logits)


# speedup vs baseline: 1.0239x; 1.0239x over previous
import functools

import jax
import jax.numpy as jnp
from jax import lax
from jax.experimental import pallas as pl
from jax.experimental.pallas import tpu as pltpu
from jax.experimental.pallas import tpu_sc as plsc

_L0 = 1000
_GROUP = 18432
_BLOCK = 2304
_MAXN = 512
_NGROUP = 8
_LANES = 16
_NROWS = 2 * _NGROUP


def _body(in_hbm, *refs):
    out0 = refs[0]
    outs = refs[1:1 + _NGROUP]
    rows_v = refs[1 + _NGROUP]
    l0_v = refs[2 + _NGROUP]
    sem_in = refs[3 + _NGROUP]
    sem_out = refs[4 + _NGROUP]
    sem_l0 = refs[5 + _NGROUP]

    wid = lax.axis_index("s") * 2 + lax.axis_index("c")

    neg_inf = jnp.full((_LANES,), -jnp.inf, dtype=jnp.float32)

    def _fire(i, _):
        t = lax.rem(i, 2)
        g = lax.div(i, 2)
        lr = wid + t * (63 - 2 * wid)
        m = lax.rem(lr, 8)
        blk = lax.div(lr, 8)
        in_off = _L0 + g * _GROUP + blk * _BLOCK + 32 * m * (m + 1)
        for k in range(8):
            @pl.when(m == k)
            def _(k=k):
                nn = 64 * (k + 1)
                pltpu.async_copy(in_hbm.at[pl.ds(in_off, nn)],
                                 rows_v.at[pl.ds(i * _MAXN, nn)], sem_in)
        return 0

    lax.fori_loop(0, _NROWS, _fire, 0)

    @pl.when(wid == 0)
    def _():
        pltpu.async_copy(in_hbm.at[pl.ds(0, _L0)], l0_v, sem_l0)

    def _pad_row(i, _):
        t = lax.rem(i, 2)
        lr = wid + t * (63 - 2 * wid)
        m = lax.rem(lr, 8)
        base = i * _MAXN

        def _pad64(c, _):
            for k in range(4):
                rows_v[pl.ds(base + c * 64 + k * _LANES, _LANES)] = neg_inf
            return 0

        lax.fori_loop(m + 1, 8, _pad64, 0)
        return 0

    lax.fori_loop(0, _NROWS, _pad_row, 0)

    pltpu.make_async_copy(in_hbm.at[pl.ds(0, _NGROUP * 576)],
                          rows_v.at[pl.ds(0, _NGROUP * 576)], sem_in).wait()

    @pl.when(wid == 0)
    def _():
        pltpu.make_async_copy(in_hbm.at[pl.ds(0, _L0)], l0_v, sem_l0).wait()
        pltpu.async_copy(l0_v, out0.at[0], sem_l0)

    for g in range(_NGROUP):
        for t in range(2):
            lr = wid + t * (63 - 2 * wid)
            pltpu.async_copy(rows_v.at[pl.ds((2 * g + t) * _MAXN, _MAXN)],
                             outs[g].at[lr], sem_out)

    pltpu.make_async_copy(in_hbm.at[pl.ds(0, _NROWS * _MAXN)],
                          rows_v, sem_out).wait()

    @pl.when(wid == 0)
    def _():
        pltpu.make_async_copy(in_hbm.at[pl.ds(0, _L0)], l0_v, sem_l0).wait()


_OUT_TYPE = (
    (jax.ShapeDtypeStruct((1, _L0), jnp.float32),)
    + tuple(jax.ShapeDtypeStruct((64, _MAXN), jnp.float32)
            for _ in range(_NGROUP))
)

_sc_interpret = functools.partial(
    pl.kernel,
    mesh=plsc.VectorSubcoreMesh(core_axis_name="c", subcore_axis_name="s"),
    out_type=_OUT_TYPE,
    scratch_types=[
        pltpu.VMEM((_NROWS * _MAXN,), jnp.float32),
        pltpu.VMEM((_L0,), jnp.float32),
        pltpu.SemaphoreType.DMA,
        pltpu.SemaphoreType.DMA,
        pltpu.SemaphoreType.DMA,
    ],
)(_body)


def kernel(logits):
    return _sc_interpret(logits)
